# Initial kernel scaffold; baseline (speedup 1.0000x reference)
import functools

import jax
import jax.numpy as jnp
from jax import lax
from jax.experimental import pallas as pl
from jax.experimental.pallas import tpu as pltpu
from jax.experimental.pallas import tpu_sc as plsc

_SUB = 128


@functools.lru_cache(maxsize=None)
def _build(V, D, N, CH):
    info = plsc.get_sparse_core_info()
    NC, NS = info.num_cores, info.num_subcores
    NW = NC * NS
    b_per_w = N // NW
    n_ch = b_per_w // CH
    n_sub = CH // _SUB
    mesh = plsc.VectorSubcoreMesh(core_axis_name="c", subcore_axis_name="s")

    @functools.partial(
        pl.kernel,
        mesh=mesh,
        out_type=jax.ShapeDtypeStruct((N, D), jnp.float32),
        scratch_types=[
            pltpu.VMEM((n_sub, _SUB), jnp.int32),
            pltpu.VMEM((CH, D), jnp.float32),
            pltpu.SemaphoreType.DMA,
        ],
    )
    def lookup(idx_hbm, table_hbm, out_hbm, idx_v, rows_v, sem):
        wid = lax.axis_index("s") * NC + lax.axis_index("c")
        base = wid * b_per_w

        def body(c, carry):
            off = base + c * CH
            pltpu.sync_copy(idx_hbm.at[pl.ds(off // _SUB, n_sub)], idx_v)
            copies = [
                pltpu.async_copy(
                    table_hbm.at[idx_v.at[j]],
                    rows_v.at[pl.ds(j * _SUB, _SUB)],
                    sem,
                )
                for j in range(n_sub)
            ]
            for cp in copies:
                cp.wait()
            pltpu.sync_copy(rows_v, out_hbm.at[pl.ds(off, CH)])
            return carry

        lax.fori_loop(0, n_ch, body, 0)

    return lookup


def kernel(x, W):
    B, S = x.shape
    V, D = W.shape
    N = B * S
    idx2d = x.reshape(N // _SUB, _SUB)
    out = _build(V, D, N, 512)(idx2d, W)
    return out.reshape(B, S, D)


# speedup vs baseline: 8.1912x; 8.1912x over previous
import functools

import jax
import jax.numpy as jnp
from jax import lax
from jax.experimental import pallas as pl
from jax.experimental.pallas import tpu as pltpu
from jax.experimental.pallas import tpu_sc as plsc

_SUB = 128


@functools.lru_cache(maxsize=None)
def _build(V, D, N, CH):
    info = plsc.get_sparse_core_info()
    NC, NS = info.num_cores, info.num_subcores
    NW = NC * NS
    b_per_w = N // NW
    n_ch = b_per_w // CH
    n_sub = CH // _SUB
    mesh = plsc.VectorSubcoreMesh(core_axis_name="c", subcore_axis_name="s")

    @functools.partial(
        pl.kernel,
        mesh=mesh,
        out_type=jax.ShapeDtypeStruct((N, D), jnp.float32),
        scratch_types=[
            pltpu.VMEM((CH,), jnp.int32),
            pltpu.VMEM((CH, D), jnp.float32),
            pltpu.SemaphoreType.DMA,
        ],
    )
    def lookup(idx_hbm, table_hbm, out_hbm, idx_v, rows_v, sem):
        wid = lax.axis_index("s") * NC + lax.axis_index("c")
        base = wid * b_per_w

        def body(c, carry):
            off = base + c * CH
            pltpu.sync_copy(idx_hbm.at[pl.ds(off, CH)], idx_v)
            copies = [
                pltpu.async_copy(
                    table_hbm.at[idx_v.at[pl.ds(j * _SUB, _SUB)]],
                    rows_v.at[pl.ds(j * _SUB, _SUB)],
                    sem,
                )
                for j in range(n_sub)
            ]
            for cp in copies:
                cp.wait()
            pltpu.sync_copy(rows_v, out_hbm.at[pl.ds(off, CH)])
            return carry

        lax.fori_loop(0, n_ch, body, 0)

    return lookup


def kernel(x, W):
    B, S = x.shape
    V, D = W.shape
    N = B * S
    idx = x.reshape(N)
    out = _build(V, D, N, 512)(idx, W)
    return out.reshape(B, S, D)


# speedup vs baseline: 9.1626x; 1.1186x over previous
import functools

import jax
import jax.numpy as jnp
from jax import lax
from jax.experimental import pallas as pl
from jax.experimental.pallas import tpu as pltpu
from jax.experimental.pallas import tpu_sc as plsc

_SUB = 128
_NBUF = 4


@functools.lru_cache(maxsize=None)
def _build(V, D, N):
    info = plsc.get_sparse_core_info()
    NC, NS = info.num_cores, info.num_subcores
    NW = NC * NS
    b_per_w = N // NW
    n_ch = b_per_w // _SUB
    n_t = n_ch // _NBUF
    mesh = plsc.VectorSubcoreMesh(core_axis_name="c", subcore_axis_name="s")

    @functools.partial(
        pl.kernel,
        mesh=mesh,
        out_type=jax.ShapeDtypeStruct((N, D), jnp.float32),
        scratch_types=[
            pltpu.VMEM((b_per_w,), jnp.int32),
            pltpu.VMEM((_NBUF, _SUB, D), jnp.float32),
            pltpu.SemaphoreType.DMA((_NBUF,)),
            pltpu.SemaphoreType.DMA((_NBUF,)),
        ],
    )
    def lookup(idx_hbm, table_hbm, out_hbm, idx_v, rows_v, sem_g, sem_s):
        wid = lax.axis_index("s") * NC + lax.axis_index("c")
        base = wid * b_per_w
        pltpu.sync_copy(idx_hbm.at[pl.ds(base, b_per_w)], idx_v)

        def gather(c, b):
            pltpu.async_copy(
                table_hbm.at[idx_v.at[pl.ds(c * _SUB, _SUB)]],
                rows_v.at[b],
                sem_g.at[b],
            )

        def store(c, b):
            pltpu.async_copy(
                rows_v.at[b],
                out_hbm.at[pl.ds(base + c * _SUB, _SUB)],
                sem_s.at[b],
            )

        def wait_store(b):
            pltpu.make_async_copy(
                rows_v.at[b],
                out_hbm.at[pl.ds(base, _SUB)],
                sem_s.at[b],
            ).wait()

        def wait_gather(b):
            pltpu.make_async_copy(
                table_hbm.at[idx_v.at[pl.ds(0, _SUB)]],
                rows_v.at[b],
                sem_g.at[b],
            ).wait()

        def body(t, carry):
            for b in range(_NBUF):

                @pl.when(t > 0)
                def _():
                    wait_store(b)

                gather(t * _NBUF + b, b)
            for b in range(_NBUF):
                wait_gather(b)
                store(t * _NBUF + b, b)
            return carry

        lax.fori_loop(0, n_t, body, 0)
        for b in range(_NBUF):
            wait_store(b)

    return lookup


def kernel(x, W):
    B, S = x.shape
    V, D = W.shape
    N = B * S
    out = _build(V, D, N)(x.reshape(N), W)
    return out.reshape(B, S, D)


# speedup vs baseline: 9.2736x; 1.0121x over previous
import functools

import jax
import jax.numpy as jnp
from jax import lax
from jax.experimental import pallas as pl
from jax.experimental.pallas import tpu as pltpu
from jax.experimental.pallas import tpu_sc as plsc

_SUB = 128
_NBUF = 4


@functools.lru_cache(maxsize=None)
def _build(V, D, N):
    info = plsc.get_sparse_core_info()
    NC, NS = info.num_cores, info.num_subcores
    NW = NC * NS
    b_per_w = N // NW
    n_ch = b_per_w // _SUB
    n_t = n_ch // _NBUF
    mesh = plsc.VectorSubcoreMesh(core_axis_name="c", subcore_axis_name="s")

    @functools.partial(
        pl.kernel,
        mesh=mesh,
        out_type=jax.ShapeDtypeStruct((N, D), jnp.float32),
        scratch_types=[
            pltpu.VMEM((b_per_w,), jnp.int32),
            pltpu.VMEM((_NBUF, _SUB, D), jnp.float32),
            pltpu.SemaphoreType.DMA((_NBUF,)),
            pltpu.SemaphoreType.DMA((_NBUF,)),
        ],
    )
    def lookup(idx_hbm, table_hbm, out_hbm, idx_v, rows_v, sem_g, sem_s):
        wid = lax.axis_index("s") * NC + lax.axis_index("c")
        base = wid * b_per_w
        pltpu.sync_copy(idx_hbm.at[pl.ds(base, b_per_w)], idx_v)

        def gather(c, b):
            pltpu.async_copy(
                table_hbm.at[idx_v.at[pl.ds(c * _SUB, _SUB)]],
                rows_v.at[b],
                sem_g.at[b],
            )

        def store(c, b):
            pltpu.async_copy(
                rows_v.at[b],
                out_hbm.at[pl.ds(base + c * _SUB, _SUB)],
                sem_s.at[b],
            )

        def wait_store(b):
            pltpu.make_async_copy(
                rows_v.at[b],
                out_hbm.at[pl.ds(base, _SUB)],
                sem_s.at[b],
            ).wait()

        def wait_gather(b):
            pltpu.make_async_copy(
                table_hbm.at[idx_v.at[pl.ds(0, _SUB)]],
                rows_v.at[b],
                sem_g.at[b],
            ).wait()

        def body(t, carry):
            for b in range(_NBUF):

                @pl.when(t > 0)
                def _():
                    wait_store(b)

                gather(t * _NBUF + b, b)
                if b == 0:

                    @pl.when(t > 0)
                    def _():
                        wait_gather(_NBUF - 1)
                        store(t * _NBUF - 1, _NBUF - 1)

                else:
                    wait_gather(b - 1)
                    store(t * _NBUF + b - 1, b - 1)
            return carry

        lax.fori_loop(0, n_t, body, 0)
        wait_gather(_NBUF - 1)
        store(n_ch - 1, _NBUF - 1)
        for b in range(_NBUF):
            wait_store(b)

    return lookup


def kernel(x, W):
    B, S = x.shape
    V, D = W.shape
    N = B * S
    out = _build(V, D, N)(x.reshape(N), W)
    return out.reshape(B, S, D)
